# baseline (speedup 1.0000x reference)
import functools

import jax
import jax.numpy as jnp
from jax import lax
from jax.experimental import pallas as pl
from jax.experimental.pallas import tpu as pltpu
from jax.experimental.pallas import tpu_sc as plsc

N, D, E = 4096, 768, 8
TM = 256
A = 2 * N
P = A + E * TM
G = P // TM
NW = 16
AW = A // NW
LANES = 16

_NEG_INF = -1e30


def _router_body(x_ref, gw_ref, gb_ref, i12_ref, cw_ref):
    xt = x_ref[...]
    logits = jax.lax.dot_general(
        xt, gw_ref[...], (((1,), (1,)), ((), ())),
        preferred_element_type=jnp.float32,
    ) + gb_ref[...]
    w = jax.nn.softmax(logits, axis=-1)
    eidx = jax.lax.broadcasted_iota(jnp.int32, w.shape, 1)
    i1 = jnp.argmax(w, axis=1)[:, None]
    v1 = jnp.max(w, axis=1, keepdims=True)
    wm = jnp.where(eidx == i1, _NEG_INF, w)
    i2 = jnp.argmax(wm, axis=1)[:, None]
    v2 = jnp.max(wm, axis=1, keepdims=True)
    r = jnp.exp(v2 - v1)
    c1 = 1.0 / (1.0 + r)
    c2 = r / (1.0 + r)
    i12_ref[...] = jnp.concatenate([i1, i2], axis=1)
    cw_ref[...] = jnp.concatenate([c1, c2], axis=1)


def _router(x, gate_W, gate_b2):
    TN = 1024
    return pl.pallas_call(
        _router_body,
        grid=(N // TN,),
        in_specs=[
            pl.BlockSpec((TN, D), lambda i: (i, 0)),
            pl.BlockSpec((E, D), lambda i: (0, 0)),
            pl.BlockSpec((1, E), lambda i: (0, 0)),
        ],
        out_specs=[
            pl.BlockSpec((TN, 2), lambda i: (i, 0)),
            pl.BlockSpec((TN, 2), lambda i: (i, 0)),
        ],
        out_shape=[
            jax.ShapeDtypeStruct((N, 2), jnp.int32),
            jax.ShapeDtypeStruct((N, 2), jnp.float32),
        ],
    )(x, gate_W, gate_b2)


def _lane_iota():
    return lax.broadcasted_iota(jnp.int32, (LANES,), 0)


def _splat_lane(vec, lane):
    s = jnp.sum(jnp.where(_lane_iota() == lane, vec, 0))
    return jnp.zeros((LANES,), vec.dtype) + s


def _ones():
    return jnp.full((LANES,), 1, jnp.int32)


def _const_vec(k):
    return jnp.full((LANES,), k, jnp.int32)


def _lane_splat(vec, lanev):
    return lax.gather(
        vec, lanev[:, None],
        dimension_numbers=lax.GatherDimensionNumbers(
            offset_dims=(), collapsed_slice_dims=(0,), start_index_map=(0,)),
        slice_sizes=(1,),
        mode=lax.GatherScatterMode.PROMISE_IN_BOUNDS)


def _dispatch_body(eid_hbm, cwf_hbm, stok_hbm, cs_hbm, inv1_hbm, inv2_hbm,
                   gid_hbm, eid_all_v, cw_v, dst_v, dst2d, tok2d, cs2d,
                   inv1_v, inv2_v, gid_v, pad2d, padv2d, tmp_v, sid_v):
    cid = lax.axis_index("c")
    sid = lax.axis_index("s")

    def splat_last(vec):
        return _lane_splat(vec, _const_vec(LANES - 1))

    def splat_lane(vec, lane):
        return _lane_splat(vec, _const_vec(lane))

    def splat_lane_dyn(vec, lanev):
        return _lane_splat(vec, lanev)

    @pl.when(cid == 0)
    def _():
        for w in range(NW):
            @pl.when(sid == w)
            def _w(w=w):
                sid_v[pl.ds(0, LANES)] = _const_vec(w)
        sidv = sid_v[pl.ds(0, LANES)]

        pltpu.sync_copy(eid_hbm, eid_all_v)
        pltpu.sync_copy(cwf_hbm.at[pl.ds(sid * AW, AW)], cw_v)

        limv = sidv * (AW // LANES)

        def hstep(c, carry):
            accs, paccs, cidx = carry
            v = eid_all_v[pl.ds(c * LANES, LANES)]
            pm = cidx < limv
            naccs, npaccs = [], []
            for e in range(E):
                sel = jnp.where(v == e, _ones(), _const_vec(0))
                naccs.append(accs[e] + sel)
                npaccs.append(paccs[e] + jnp.where(pm, sel, _const_vec(0)))
            return tuple(naccs), tuple(npaccs), cidx + _ones()

        zero8 = tuple(jnp.zeros((LANES,), jnp.int32) for _ in range(E))
        accs, paccs, _ = lax.fori_loop(
            0, A // LANES, hstep,
            (zero8, zero8, jnp.zeros((LANES,), jnp.int32)))
        totv = jnp.zeros((LANES,), jnp.int32)
        for e in range(E):
            totv = jnp.where(_lane_iota() == e,
                             splat_last(jnp.cumsum(accs[e])), totv)
        padded = jnp.right_shift(totv + (TM - 1), 8) << 8
        segsum = jnp.cumsum(padded)
        seg_start = segsum - padded
        prefs = [splat_last(jnp.cumsum(paccs[e])) for e in range(E)]

        @pl.when(sid == 0)
        def _gid():
            endtile = jnp.right_shift(segsum, 8)
            for c in range(3):
                gvec = _lane_iota() + c * LANES
                acc = jnp.zeros((LANES,), jnp.int32)
                for ep in range(E - 1):
                    et = splat_lane(endtile, ep)
                    acc = acc + jnp.where(gvec >= et, _ones(), _const_vec(0))
                gid_v[pl.ds(c * LANES, LANES)] = acc
            pltpu.sync_copy(gid_v, gid_hbm)

        @pl.when(sid < E)
        def _pads():
            sb = splat_lane_dyn(seg_start, sidv)
            ct = splat_lane_dyn(totv, sidv)
            pd = splat_lane_dyn(padded, sidv)
            base = sb + ct
            lim = sb + pd
            for q in range(4):
                for c4 in range(4):
                    pos = base + _const_vec(q * 64 + c4 * LANES) + _lane_iota()
                    dump = _const_vec(P) + sidv * LANES + _lane_iota()
                    idx = jnp.where(pos < lim, pos, dump)
                    pad2d[q, pl.ds(c4 * LANES, LANES)] = idx
                    padv2d[q, pl.ds(c4 * LANES, LANES)] = jnp.zeros(
                        (LANES,), jnp.int32)
            for q in range(4):
                pltpu.sync_copy(padv2d.at[q], stok_hbm.at[pad2d.at[q]])

        @pl.when(sid >= E)
        def _tail():
            ts = splat_lane(segsum, E - 1)
            base = ts + (sidv - _const_vec(E)) * _const_vec(E * TM // E)
            lim = _const_vec(P)
            for q in range(4):
                for c4 in range(4):
                    pos = base + _const_vec(q * 64 + c4 * LANES) + _lane_iota()
                    dump = _const_vec(P) + sidv * LANES + _lane_iota()
                    idx = jnp.where(pos < lim, pos, dump)
                    pad2d[q, pl.ds(c4 * LANES, LANES)] = idx
                    padv2d[q, pl.ds(c4 * LANES, LANES)] = jnp.zeros(
                        (LANES,), jnp.int32)
            for q in range(4):
                pltpu.sync_copy(padv2d.at[q], stok_hbm.at[pad2d.at[q]])

        for e in range(E):
            bs0 = splat_lane(seg_start, e) + prefs[e]
            def cbody(c, bs, e=e):
                v = eid_all_v[pl.ds(sid * AW + c * LANES, LANES)]
                m = v == e
                mi = jnp.where(m, _ones(), _const_vec(0))
                csum = jnp.cumsum(mi)
                cur = dst_v[pl.ds(c * LANES, LANES)]
                dst_v[pl.ds(c * LANES, LANES)] = jnp.where(
                    m, bs + csum - 1, cur)
                return bs + splat_last(csum)
            lax.fori_loop(0, AW // LANES, cbody, bs0)

        for c in range(AW // LANES):
            r, off = c // 4, (c % 4) * LANES
            dv = dst_v[pl.ds(c * LANES, LANES)]
            dst2d[r, pl.ds(off, LANES)] = dv
            jv = sidv * AW + _const_vec(c * LANES) + _lane_iota()
            tok2d[r, pl.ds(off, LANES)] = jnp.right_shift(jv, 1)
            cs2d[r, pl.ds(off, LANES)] = cw_v[pl.ds(c * LANES, LANES)]
        for r in range(AW // 64):
            pltpu.sync_copy(tok2d.at[r], stok_hbm.at[dst2d.at[r]])
            pltpu.sync_copy(cs2d.at[r], cs_hbm.at[dst2d.at[r]])

        for tc in range(AW // 2 // LANES):
            idx = (_const_vec(tc * LANES) + _lane_iota()) * 2
            v1 = plsc.load_gather(dst_v, [idx])
            v2 = plsc.load_gather(dst_v, [idx + _ones()])
            r, off = tc // 4, (tc % 4) * LANES
            inv1_v[r, pl.ds(off, LANES)] = v1
            inv2_v[r, pl.ds(off, LANES)] = v2
        nrow = AW // 2 // 64
        pltpu.sync_copy(inv1_v, inv1_hbm.at[pl.ds(sid * nrow, nrow)])
        pltpu.sync_copy(inv2_v, inv2_hbm.at[pl.ds(sid * nrow, nrow)])


def _dispatch(eid_flat, cw_flat):
    mesh = plsc.VectorSubcoreMesh(core_axis_name="c", subcore_axis_name="s")
    f = pl.kernel(
        _dispatch_body, mesh=mesh,
        compiler_params=pltpu.CompilerParams(needs_layout_passes=False),
        out_type=[
            jax.ShapeDtypeStruct((P + NW * LANES,), jnp.int32),
            jax.ShapeDtypeStruct((P,), jnp.float32),
            jax.ShapeDtypeStruct((N // 64, 64), jnp.int32),
            jax.ShapeDtypeStruct((N // 64, 64), jnp.int32),
            jax.ShapeDtypeStruct((48,), jnp.int32),
        ],
        scratch_types=[
            pltpu.VMEM((A,), jnp.int32),
            pltpu.VMEM((AW,), jnp.float32),
            pltpu.VMEM((AW,), jnp.int32),
            pltpu.VMEM((AW // 64, 64), jnp.int32),
            pltpu.VMEM((AW // 64, 64), jnp.int32),
            pltpu.VMEM((AW // 64, 64), jnp.float32),
            pltpu.VMEM((AW // 2 // 64, 64), jnp.int32),
            pltpu.VMEM((AW // 2 // 64, 64), jnp.int32),
            pltpu.VMEM((48,), jnp.int32),
            pltpu.VMEM((4, 64), jnp.int32),
            pltpu.VMEM((4, 64), jnp.int32),
            pltpu.VMEM((LANES,), jnp.int32),
            pltpu.VMEM((LANES,), jnp.int32),
        ],
    )
    return f(eid_flat, cw_flat)


def _gather_body(x_hbm, stok_hbm, xg_hbm, idx_v, row_v, sem):
    cid = lax.axis_index("c")
    sid = lax.axis_index("s")
    wid = sid * 2 + cid
    rows_per_w = P // 64 // 32
    for j in range(rows_per_w):
        rrow = wid * rows_per_w + j
        pltpu.sync_copy(stok_hbm.at[rrow], idx_v)
        pltpu.async_copy(x_hbm.at[idx_v], row_v, sem).wait()
        pltpu.sync_copy(row_v, xg_hbm.at[pl.ds(rrow * 64, 64), :])


def _gather(x, stok2d):
    mesh = plsc.VectorSubcoreMesh(core_axis_name="c", subcore_axis_name="s")
    f = pl.kernel(
        _gather_body, mesh=mesh,
        compiler_params=pltpu.CompilerParams(needs_layout_passes=False),
        out_type=jax.ShapeDtypeStruct((P, D), jnp.float32),
        scratch_types=[
            pltpu.VMEM((64,), jnp.int32),
            pltpu.VMEM((64, D), jnp.float32),
            pltpu.SemaphoreType.DMA,
        ],
    )
    return f(x, stok2d)


def _matmul_body(gid_ref, xg_ref, ew_ref, eb_ref, cs_ref, ys_ref):
    y = jax.lax.dot_general(
        xg_ref[...], ew_ref[0], (((1,), (1,)), ((), ())),
        preferred_element_type=jnp.float32,
    ) + eb_ref[0]
    ys_ref[...] = y * cs_ref[...]


def _grouped_matmul(gid, xg, expert_W, expert_b, cs2):
    grid_spec = pltpu.PrefetchScalarGridSpec(
        num_scalar_prefetch=1,
        grid=(G,),
        in_specs=[
            pl.BlockSpec((TM, D), lambda g, gid: (g, 0)),
            pl.BlockSpec((1, D, D), lambda g, gid: (gid[g], 0, 0)),
            pl.BlockSpec((1, 1, D), lambda g, gid: (gid[g], 0, 0)),
            pl.BlockSpec((TM, 1), lambda g, gid: (g, 0)),
        ],
        out_specs=pl.BlockSpec((TM, D), lambda g, gid: (g, 0)),
    )
    return pl.pallas_call(
        _matmul_body,
        grid_spec=grid_spec,
        out_shape=jax.ShapeDtypeStruct((P, D), jnp.float32),
        compiler_params=pltpu.CompilerParams(
            dimension_semantics=("arbitrary",),
        ),
    )(gid, xg, expert_W, expert_b.reshape(E, 1, D), cs2)


def _combine_body(ys_hbm, inv1_hbm, inv2_hbm, out_hbm, i1_v, i2_v,
                  yb1, yb2, sem):
    cid = lax.axis_index("c")
    sid = lax.axis_index("s")
    wid = sid * 2 + cid
    for ch in range(2):
        row = wid * 2 + ch
        pltpu.sync_copy(inv1_hbm.at[row], i1_v)
        pltpu.sync_copy(inv2_hbm.at[row], i2_v)
        pltpu.async_copy(ys_hbm.at[i1_v], yb1, sem).wait()
        pltpu.async_copy(ys_hbm.at[i2_v], yb2, sem).wait()

        def abody(t, _):
            for v in range(D // LANES):
                a = yb1[t, pl.ds(v * LANES, LANES)]
                b = yb2[t, pl.ds(v * LANES, LANES)]
                yb1[t, pl.ds(v * LANES, LANES)] = a + b
            return 0
        lax.fori_loop(0, 64, abody, 0)
        pltpu.sync_copy(yb1, out_hbm.at[pl.ds(row * 64, 64), :])


def _combine(ys, inv1, inv2):
    mesh = plsc.VectorSubcoreMesh(core_axis_name="c", subcore_axis_name="s")
    f = pl.kernel(
        _combine_body, mesh=mesh,
        compiler_params=pltpu.CompilerParams(needs_layout_passes=False),
        out_type=jax.ShapeDtypeStruct((N, D), jnp.float32),
        scratch_types=[
            pltpu.VMEM((64,), jnp.int32),
            pltpu.VMEM((64,), jnp.int32),
            pltpu.VMEM((64, D), jnp.float32),
            pltpu.VMEM((64, D), jnp.float32),
            pltpu.SemaphoreType.DMA,
        ],
    )
    return f(ys, inv1, inv2)


def kernel(x, gate_W, gate_b, expert_W, expert_b):
    gate_b2 = gate_b.reshape(1, E)
    i12, cw = _router(x, gate_W, gate_b2)
    stok, cs, inv1, inv2, gid = _dispatch(i12.reshape(-1), cw.reshape(-1))
    xg = _gather(x, stok[:P].reshape(P // 64, 64))
    ys = _grouped_matmul(gid, xg, expert_W, expert_b.reshape(E, 1, D), cs.reshape(P, 1))
    return _combine(ys, inv1, inv2)


# speedup vs baseline: 7.6998x; 7.6998x over previous
import functools

import jax
import jax.numpy as jnp
from jax.experimental import pallas as pl
from jax.experimental.pallas import tpu as pltpu

_NEG_INF = -1e30


def _moe_body(x_ref, gw_ref, gb_ref, ew_ref, eb_ref, out_ref, *, n_experts):
    xt = x_ref[...]
    logits = jax.lax.dot_general(
        xt, gw_ref[...], (((1,), (1,)), ((), ())),
        preferred_element_type=jnp.float32,
    ) + gb_ref[...]
    w = jax.nn.softmax(logits, axis=-1)
    eidx = jax.lax.broadcasted_iota(jnp.int32, w.shape, 1)
    i1 = jnp.argmax(w, axis=1)[:, None]
    v1 = jnp.max(w, axis=1, keepdims=True)
    wm = jnp.where(eidx == i1, _NEG_INF, w)
    i2 = jnp.argmax(wm, axis=1)[:, None]
    v2 = jnp.max(wm, axis=1, keepdims=True)
    r = jnp.exp(v2 - v1)
    c1 = 1.0 / (1.0 + r)
    c2 = r / (1.0 + r)
    combine = jnp.where(eidx == i1, c1, 0.0) + jnp.where(eidx == i2, c2, 0.0)
    acc = jax.lax.dot_general(
        combine, eb_ref[...], (((1,), (0,)), ((), ())),
        preferred_element_type=jnp.float32,
    )
    for e in range(n_experts):
        y = jax.lax.dot_general(
            xt, ew_ref[e], (((1,), (1,)), ((), ())),
            preferred_element_type=jnp.float32,
        )
        c_e = jnp.where(i1 == e, c1, jnp.where(i2 == e, c2, 0.0))
        acc = acc + c_e * y
    out_ref[...] = acc


def kernel(x, gate_W, gate_b, expert_W, expert_b):
    N, d = x.shape
    E = gate_W.shape[0]
    TN = 512
    grid = (N // TN,)
    gate_b2 = gate_b.reshape(1, E)
    return pl.pallas_call(
        functools.partial(_moe_body, n_experts=E),
        grid=grid,
        in_specs=[
            pl.BlockSpec((TN, d), lambda i: (i, 0)),
            pl.BlockSpec((E, d), lambda i: (0, 0)),
            pl.BlockSpec((1, E), lambda i: (0, 0)),
            pl.BlockSpec((E, d, d), lambda i: (0, 0, 0)),
            pl.BlockSpec((E, d), lambda i: (0, 0)),
        ],
        out_specs=pl.BlockSpec((TN, d), lambda i: (i, 0)),
        out_shape=jax.ShapeDtypeStruct((N, d), x.dtype),
        compiler_params=pltpu.CompilerParams(
            dimension_semantics=("arbitrary",),
        ),
    )(x, gate_W, gate_b2, expert_W, expert_b)
